# baseline (speedup 1.0000x reference)
import jax
import jax.numpy as jnp
from jax.experimental import pallas as pl


def _body(z_ref, w_ref, u_ref, out_ref):
    z = z_ref[...]
    w = w_ref[:, 1:-1] + 1e-05
    u = u_ref[...]
    R, S = z.shape
    K = S - 1

    zmid = 0.5 * (z[:, :-1] + z[:, 1:])

    kk = jax.lax.broadcasted_iota(jnp.int32, (S - 2, K), 0)
    jj = jax.lax.broadcasted_iota(jnp.int32, (S - 2, K), 1)
    tri = (kk < jj).astype(jnp.float32)
    cs = jax.lax.dot(w, tri, preferred_element_type=jnp.float32)
    total = cs[:, -1:]
    cdf = cs / total

    zpad = jnp.zeros((R, 1), jnp.float32)
    d0 = jnp.concatenate([zpad, cdf[:, 1:] - cdf[:, :-1]], axis=1)
    b0 = jnp.concatenate([zmid[:, :1], zmid[:, 1:] - zmid[:, :-1]], axis=1)
    d1 = jnp.concatenate([cdf[:, 1:] - cdf[:, :-1], zpad], axis=1)
    b1 = jnp.concatenate([zmid[:, 1:] - zmid[:, :-1], zpad], axis=1)

    acc_c0 = jnp.zeros_like(u)
    acc_b0 = jnp.zeros_like(u)
    acc_c1 = jnp.zeros_like(u)
    acc_b1 = jnp.broadcast_to(zmid[:, :1], u.shape)
    for k in range(K):
        m = cdf[:, k:k + 1] <= u
        acc_c0 = acc_c0 + jnp.where(m, d0[:, k:k + 1], 0.0)
        acc_b0 = acc_b0 + jnp.where(m, b0[:, k:k + 1], 0.0)
        acc_c1 = acc_c1 + jnp.where(m, d1[:, k:k + 1], 0.0)
        acc_b1 = acc_b1 + jnp.where(m, b1[:, k:k + 1], 0.0)

    denom = acc_c1 - acc_c0
    denom = jnp.where(denom < 1e-05, 1.0, denom)
    t = (u - acc_c0) / denom
    out_ref[...] = acc_b0 + t * (acc_b1 - acc_b0)


def kernel(rays_o, rays_d, z_vals, weights, u, num_importance):
    del rays_o, rays_d, num_importance
    N, S = z_vals.shape
    NI = u.shape[1]
    R = 128
    grid = (N // R,)
    return pl.pallas_call(
        _body,
        grid=grid,
        in_specs=[
            pl.BlockSpec((R, S), lambda i: (i, 0)),
            pl.BlockSpec((R, S), lambda i: (i, 0)),
            pl.BlockSpec((R, NI), lambda i: (i, 0)),
        ],
        out_specs=pl.BlockSpec((R, NI), lambda i: (i, 0)),
        out_shape=jax.ShapeDtypeStruct((N, NI), jnp.float32),
    )(z_vals, weights, u)


# speedup vs baseline: 4.7572x; 4.7572x over previous
import functools

import jax
import jax.numpy as jnp
from jax import lax
from jax.experimental import pallas as pl
from jax.experimental.pallas import tpu as pltpu
from jax.experimental.pallas import tpu_sc as plsc

_L = 16


def _prep_body(z_ref, w_ref, cdf_ref, zmid_ref):
    z = z_ref[...]
    w = w_ref[:, 1:-1] + 1e-05
    R, S = z.shape
    K = S - 1

    kk = lax.broadcasted_iota(jnp.int32, (S - 2, S), 0)
    jj = lax.broadcasted_iota(jnp.int32, (S - 2, S), 1)
    tri = jnp.where(kk < jnp.minimum(jj, K), 1.0, 0.0)
    cs = lax.dot(w, tri, preferred_element_type=jnp.float32)
    cdf_ref[...] = cs / cs[:, -1:]

    zmid = 0.5 * (z[:, :-1] + z[:, 1:])
    zmid_ref[...] = jnp.concatenate([zmid, zmid[:, -1:]], axis=1)


def _sc_body(num_rays, chunk, cdf_hbm, zmid_hbm, u_hbm, out_hbm,
             cdf_v, zmid_v, u_v, out_v):
    nc = 2
    wid = lax.axis_index("s") * nc + lax.axis_index("c")
    nw = 32
    per_w = num_rays // nw
    base = wid * per_w
    ni = u_v.shape[1]
    kmax = cdf_v.shape[1] - 1

    def ray_body(r, carry):
        rvec = jnp.broadcast_to(r, (_L,)).astype(jnp.int32)
        for j in range(ni // _L):
            uu = u_v[r, pl.ds(j * _L, _L)]
            pos = jnp.zeros((_L,), jnp.int32)
            for w in (128, 64, 32, 16, 8, 4, 2, 1):
                cand = pos + w
                safe = jnp.minimum(cand, kmax) - 1
                c = plsc.load_gather(cdf_v, [rvec, safe])
                ok = (cand <= kmax) & (c <= uu)
                pos = jnp.where(ok, cand, pos)
            below = jnp.maximum(pos - 1, 0)
            above = jnp.minimum(pos, kmax - 1)
            c0 = plsc.load_gather(cdf_v, [rvec, below])
            c1 = plsc.load_gather(cdf_v, [rvec, above])
            b0 = plsc.load_gather(zmid_v, [rvec, below])
            b1 = plsc.load_gather(zmid_v, [rvec, above])
            denom = c1 - c0
            denom = jnp.where(denom < 1e-05, 1.0, denom)
            t = (uu - c0) / denom
            out_v[r, pl.ds(j * _L, _L)] = b0 + t * (b1 - b0)
        return carry

    def chunk_body(i, carry):
        off = base + i * chunk
        pltpu.sync_copy(cdf_hbm.at[pl.ds(off, chunk)], cdf_v)
        pltpu.sync_copy(zmid_hbm.at[pl.ds(off, chunk)], zmid_v)
        pltpu.sync_copy(u_hbm.at[pl.ds(off, chunk)], u_v)
        lax.fori_loop(0, chunk, ray_body, 0)
        pltpu.sync_copy(out_v, out_hbm.at[pl.ds(off, chunk)])
        return carry

    lax.fori_loop(0, per_w // chunk, chunk_body, 0)


def kernel(rays_o, rays_d, z_vals, weights, u, num_importance):
    del rays_o, rays_d, num_importance
    N, S = z_vals.shape
    NI = u.shape[1]
    R = 512
    cdf, zmid = pl.pallas_call(
        _prep_body,
        grid=(N // R,),
        in_specs=[
            pl.BlockSpec((R, S), lambda i: (i, 0)),
            pl.BlockSpec((R, S), lambda i: (i, 0)),
        ],
        out_specs=[
            pl.BlockSpec((R, S), lambda i: (i, 0)),
            pl.BlockSpec((R, S), lambda i: (i, 0)),
        ],
        out_shape=[
            jax.ShapeDtypeStruct((N, S), jnp.float32),
            jax.ShapeDtypeStruct((N, S), jnp.float32),
        ],
    )(z_vals, weights)

    chunk = 64
    mesh = plsc.VectorSubcoreMesh(core_axis_name="c", subcore_axis_name="s")
    sc = functools.partial(
        pl.kernel,
        out_type=jax.ShapeDtypeStruct((N, NI), jnp.float32),
        mesh=mesh,
        scratch_types=[
            pltpu.VMEM((chunk, S), jnp.float32),
            pltpu.VMEM((chunk, S), jnp.float32),
            pltpu.VMEM((chunk, NI), jnp.float32),
            pltpu.VMEM((chunk, NI), jnp.float32),
        ],
        compiler_params=pltpu.CompilerParams(
            use_tc_tiling_on_sc=False, needs_layout_passes=False),
    )(functools.partial(_sc_body, N, chunk))
    return sc(cdf, zmid, u)


# speedup vs baseline: 4.9285x; 1.0360x over previous
import functools

import jax
import jax.numpy as jnp
from jax import lax
from jax.experimental import pallas as pl
from jax.experimental.pallas import tpu as pltpu
from jax.experimental.pallas import tpu_sc as plsc

_L = 16


def _prep_body(w_ref, cdf_ref):
    w = w_ref[:, 1:-1] + 1e-05
    S = w_ref.shape[1]
    kk = lax.broadcasted_iota(jnp.int32, (S - 2, S), 0)
    jj = lax.broadcasted_iota(jnp.int32, (S - 2, S), 1)
    tri = jnp.where(kk <= jj - 2, 1.0, 0.0)
    cs = lax.dot(w, tri, preferred_element_type=jnp.float32)
    cdf_ref[...] = cs / cs[:, -1:]


def _sc_ray(cdf_v, z_v, u_v, out_v, kmax, r, rvec, j):
    uu = u_v[r, pl.ds(j * _L, _L)]
    pos = jnp.zeros((_L,), jnp.int32)
    for w in (128, 64, 32, 16, 8, 4, 2, 1):
        cand = pos + w
        idx = jnp.minimum(cand, kmax)
        c = plsc.load_gather(cdf_v, [rvec, idx])
        pos = jnp.where(c <= uu, cand, pos)
    pp = jnp.minimum(pos + 1, kmax)
    bb = jnp.minimum(pos, kmax - 1)
    c0 = plsc.load_gather(cdf_v, [rvec, pos])
    c1 = plsc.load_gather(cdf_v, [rvec, pp])
    za = plsc.load_gather(z_v, [rvec, pos - 1])
    zb = plsc.load_gather(z_v, [rvec, pos])
    zc = plsc.load_gather(z_v, [rvec, bb])
    zd = plsc.load_gather(z_v, [rvec, bb + 1])
    s0 = za + zb
    s1 = zc + zd
    denom = c1 - c0
    denom = jnp.where(denom < 1e-05, 1.0, denom)
    t = (uu - c0) / denom
    out_v[r, pl.ds(j * _L, _L)] = 0.5 * (s0 + t * (s1 - s0))


def _sc_body(num_rays, chunk, cdf_hbm, z_hbm, u_hbm, out_hbm,
             cdf_v, z_v, u_v, out_v):
    nc = 2
    wid = lax.axis_index("s") * nc + lax.axis_index("c")
    nw = 32
    per_w = num_rays // nw
    base = wid * per_w
    ni = u_v.shape[1]
    kmax = cdf_v.shape[1] - 1

    def pair_body(p, carry):
        r0 = p * 2
        r1 = r0 + 1
        rv0 = jnp.broadcast_to(r0, (_L,)).astype(jnp.int32)
        rv1 = rv0 + 1
        for j in range(ni // _L):
            _sc_ray(cdf_v, z_v, u_v, out_v, kmax, r0, rv0, j)
            _sc_ray(cdf_v, z_v, u_v, out_v, kmax, r1, rv1, j)
        return carry

    def chunk_body(i, carry):
        off = base + i * chunk
        pltpu.sync_copy(cdf_hbm.at[pl.ds(off, chunk)], cdf_v)
        pltpu.sync_copy(z_hbm.at[pl.ds(off, chunk)], z_v)
        pltpu.sync_copy(u_hbm.at[pl.ds(off, chunk)], u_v)
        lax.fori_loop(0, chunk // 2, pair_body, 0)
        pltpu.sync_copy(out_v, out_hbm.at[pl.ds(off, chunk)])
        return carry

    lax.fori_loop(0, per_w // chunk, chunk_body, 0)


def kernel(rays_o, rays_d, z_vals, weights, u, num_importance):
    del rays_o, rays_d, num_importance
    N, S = z_vals.shape
    NI = u.shape[1]
    R = 512
    cdf = pl.pallas_call(
        _prep_body,
        grid=(N // R,),
        in_specs=[pl.BlockSpec((R, S), lambda i: (i, 0))],
        out_specs=pl.BlockSpec((R, S), lambda i: (i, 0)),
        out_shape=jax.ShapeDtypeStruct((N, S), jnp.float32),
    )(weights)

    chunk = 64
    mesh = plsc.VectorSubcoreMesh(core_axis_name="c", subcore_axis_name="s")
    sc = functools.partial(
        pl.kernel,
        out_type=jax.ShapeDtypeStruct((N, NI), jnp.float32),
        mesh=mesh,
        scratch_types=[
            pltpu.VMEM((chunk, S), jnp.float32),
            pltpu.VMEM((chunk, S), jnp.float32),
            pltpu.VMEM((chunk, NI), jnp.float32),
            pltpu.VMEM((chunk, NI), jnp.float32),
        ],
        compiler_params=pltpu.CompilerParams(
            use_tc_tiling_on_sc=False, needs_layout_passes=False),
    )(functools.partial(_sc_body, N, chunk))
    return sc(cdf, z_vals, u)
